# Initial kernel scaffold; baseline (speedup 1.0000x reference)
import jax
import jax.numpy as jnp
from jax.experimental import pallas as pl


def kernel(word, pos1, pos2, word_table, pos1_table, pos2_table):
    raise NotImplementedError("write your pallas kernel here")


# speedup vs baseline: 3.3889x; 3.3889x over previous
import functools

import jax
import jax.numpy as jnp
from jax import lax
from jax.experimental import pallas as pl
from jax.experimental.pallas import tpu as pltpu
from jax.experimental.pallas import tpu_sc as plsc

_B, _L = 4096, 200
_WDIM, _PDIM = 64, 5
_FDIM = _WDIM + 2 * _PDIM
_N = _B * _L
_NW = 32
_CHUNK = 128
_SPAN = 4
_ROWS_STEP = _CHUNK * _SPAN
_STEPS = _N // (_NW * _ROWS_STEP)
_G = _N // _ROWS_STEP


def _body(word_h, pos1_h, pos2_h, wtab_h, p1tab_h, p2tab_h, out_h,
          widx_v, p1idx_v, p2idx_v, outbuf_v, wordbuf_v, p1tab_v, p2tab_v,
          sem):
    wid = lax.axis_index("s") * 2 + lax.axis_index("c")
    pltpu.sync_copy(p1tab_h, p1tab_v)
    pltpu.sync_copy(p2tab_h, p2tab_v)

    def step(s, carry):
        g = wid * _STEPS + s
        pltpu.sync_copy(word_h.at[g], widx_v)
        pltpu.sync_copy(pos1_h.at[g], p1idx_v)
        pltpu.sync_copy(pos2_h.at[g], p2idx_v)
        copies = [
            pltpu.async_copy(wtab_h.at[widx_v.at[c]],
                             wordbuf_v.at[c], sem)
            for c in range(_SPAN)
        ]
        def pos_chunk(c, carry2):
            cs = jnp.full((16,), c, jnp.int32)
            for j in range(_CHUNK // 16):
                rows = lax.iota(jnp.int32, 16) + (j * 16)
                i1 = p1idx_v[c, pl.ds(j * 16, 16)]
                i2 = p2idx_v[c, pl.ds(j * 16, 16)]
                f1 = i1 * _PDIM
                f2 = i2 * _PDIM
                for t in range(_PDIM):
                    v1 = plsc.load_gather(p1tab_v, [f1 + t])
                    plsc.store_scatter(
                        outbuf_v,
                        [cs, rows, jnp.full((16,), _WDIM + t, jnp.int32)], v1)
                    v2 = plsc.load_gather(p2tab_v, [f2 + t])
                    plsc.store_scatter(
                        outbuf_v,
                        [cs, rows,
                         jnp.full((16,), _WDIM + _PDIM + t, jnp.int32)], v2)
            return carry2
        lax.fori_loop(0, _SPAN, pos_chunk, 0)
        for cp in copies:
            cp.wait()
        def word_chunk(c, carry2):
            def word_row(r, carry3):
                for q in range(_WDIM // 16):
                    outbuf_v[c, r, pl.ds(q * 16, 16)] = (
                        wordbuf_v[c, r, pl.ds(q * 16, 16)])
                return carry3
            return lax.fori_loop(0, _CHUNK, word_row, carry2)
        lax.fori_loop(0, _SPAN, word_chunk, 0)
        pltpu.sync_copy(outbuf_v, out_h.at[g])
        return carry

    lax.fori_loop(0, _STEPS, step, 0)


def kernel(word, pos1, pos2, word_table, pos1_table, pos2_table):
    word3 = jnp.asarray(word, jnp.int32).reshape(_G, _SPAN, _CHUNK)
    pos13 = jnp.asarray(pos1, jnp.int32).reshape(_G, _SPAN, _CHUNK)
    pos23 = jnp.asarray(pos2, jnp.int32).reshape(_G, _SPAN, _CHUNK)

    mesh = plsc.VectorSubcoreMesh(core_axis_name="c", subcore_axis_name="s")
    f = pl.kernel(
        _body,
        out_type=jax.ShapeDtypeStruct((_G, _SPAN, _CHUNK, _FDIM), jnp.float32),
        mesh=mesh,
        compiler_params=pltpu.CompilerParams(
            needs_layout_passes=False, use_tc_tiling_on_sc=False),
        scratch_types=[
            pltpu.VMEM((_SPAN, _CHUNK), jnp.int32),
            pltpu.VMEM((_SPAN, _CHUNK), jnp.int32),
            pltpu.VMEM((_SPAN, _CHUNK), jnp.int32),
            pltpu.VMEM((_SPAN, _CHUNK, _FDIM), jnp.float32),
            pltpu.VMEM((_SPAN, _CHUNK, _WDIM), jnp.float32),
            pltpu.VMEM((2 * _L * _PDIM,), jnp.float32),
            pltpu.VMEM((2 * _L * _PDIM,), jnp.float32),
            pltpu.SemaphoreType.DMA,
        ],
    )
    out = f(word3, pos13, pos23, word_table,
            pos1_table.reshape(-1), pos2_table.reshape(-1))
    return out.reshape(_B, _L, _FDIM)


# speedup vs baseline: 4.3001x; 1.2689x over previous
import functools

import jax
import jax.numpy as jnp
from jax import lax
from jax.experimental import pallas as pl
from jax.experimental.pallas import tpu as pltpu
from jax.experimental.pallas import tpu_sc as plsc

_B, _L = 4096, 200
_WDIM, _PDIM = 64, 5
_FDIM = _WDIM + 2 * _PDIM
_N = _B * _L
_NW = 32
_CHUNK = 128
_SPAN = 2
_ROWS_STEP = _CHUNK * _SPAN
_STEPS = _N // (_NW * _ROWS_STEP)
_G = _N // _ROWS_STEP
_HALF = _STEPS // 2


def _body(idx_h, wtab_h, p1tab_h, p2tab_h, out_h,
          ibuf0_v, ibuf1_v, wbuf0_v, wbuf1_v, pbuf0_v, pbuf1_v,
          p1tab_v, p2tab_v, sem_g, sem_o):
    wid = lax.axis_index("s") * 2 + lax.axis_index("c")
    pltpu.sync_copy(p1tab_h, p1tab_v)
    pltpu.sync_copy(p2tab_h, p2tab_v)

    bufs = ((ibuf0_v, wbuf0_v, pbuf0_v, 0), (ibuf1_v, wbuf1_v, pbuf1_v, 1))

    def fire_gathers(g, b):
        ibuf_v, wbuf_v, _, k = bufs[b]
        pltpu.sync_copy(idx_h.at[g], ibuf_v)
        for c in range(_SPAN):
            pltpu.async_copy(wtab_h.at[ibuf_v.at[0, c]],
                             wbuf_v.at[c], sem_g.at[k])

    def drain_gathers(b):
        ibuf_v, wbuf_v, _, k = bufs[b]
        for c in range(_SPAN):
            pltpu.make_async_copy(wtab_h.at[ibuf_v.at[0, c]],
                                  wbuf_v.at[c], sem_g.at[k]).wait()

    def pos_assemble(b):
        ibuf_v, _, pbuf_v, _ = bufs[b]
        for c in range(_SPAN):
            cs = jnp.full((16,), c, jnp.int32)
            for j in range(_CHUNK // 16):
                rows = lax.iota(jnp.int32, 16) + (j * 16)
                f1 = ibuf_v[1, c, pl.ds(j * 16, 16)] * _PDIM
                f2 = ibuf_v[2, c, pl.ds(j * 16, 16)] * _PDIM
                for t in range(_PDIM):
                    v1 = plsc.load_gather(p1tab_v, [f1 + t])
                    plsc.store_scatter(
                        pbuf_v,
                        [cs, rows, jnp.full((16,), t, jnp.int32)], v1)
                    v2 = plsc.load_gather(p2tab_v, [f2 + t])
                    plsc.store_scatter(
                        pbuf_v,
                        [cs, rows,
                         jnp.full((16,), _PDIM + t, jnp.int32)], v2)

    def wait_out(g, b):
        _, wbuf_v, pbuf_v, k = bufs[b]
        for c in range(_SPAN):
            pltpu.make_async_copy(wbuf_v.at[c], out_h.at[g, c, :, 0:_WDIM],
                                  sem_o.at[k]).wait()
            pltpu.make_async_copy(pbuf_v.at[c],
                                  out_h.at[g, c, :, _WDIM:_FDIM],
                                  sem_o.at[k]).wait()

    def write_out_async(g, b):
        _, wbuf_v, pbuf_v, k = bufs[b]
        for c in range(_SPAN):
            pltpu.async_copy(wbuf_v.at[c], out_h.at[g, c, :, 0:_WDIM],
                             sem_o.at[k])
            pltpu.async_copy(pbuf_v.at[c], out_h.at[g, c, :, _WDIM:_FDIM],
                             sem_o.at[k])

    def write_out_sync(g, b):
        _, wbuf_v, pbuf_v, _ = bufs[b]
        for c in range(_SPAN):
            pltpu.sync_copy(wbuf_v.at[c], out_h.at[g, c, :, 0:_WDIM])
            pltpu.sync_copy(pbuf_v.at[c], out_h.at[g, c, :, _WDIM:_FDIM])

    g0 = wid * _STEPS
    fire_gathers(g0, 0)

    def pair(i, carry):
        g = g0 + 2 * i

        @pl.when(i >= 1)
        def _():
            wait_out(g, 0)
        drain_gathers(0)
        fire_gathers(g + 1, 1)
        pos_assemble(0)

        @pl.when(i < _HALF - 1)
        def _():
            write_out_async(g, 0)

        @pl.when(i == _HALF - 1)
        def _():
            write_out_sync(g, 0)

        @pl.when(i >= 1)
        def _():
            wait_out(g + 1, 1)
        drain_gathers(1)

        @pl.when(i < _HALF - 1)
        def _():
            fire_gathers(g + 2, 0)
        pos_assemble(1)

        @pl.when(i < _HALF - 1)
        def _():
            write_out_async(g + 1, 1)

        @pl.when(i == _HALF - 1)
        def _():
            write_out_sync(g + 1, 1)

        return carry

    lax.fori_loop(0, _HALF, pair, 0)


def kernel(word, pos1, pos2, word_table, pos1_table, pos2_table):
    idx = jnp.stack(
        [jnp.asarray(word, jnp.int32).reshape(_G, _SPAN, _CHUNK),
         jnp.asarray(pos1, jnp.int32).reshape(_G, _SPAN, _CHUNK),
         jnp.asarray(pos2, jnp.int32).reshape(_G, _SPAN, _CHUNK)],
        axis=1)

    mesh = plsc.VectorSubcoreMesh(core_axis_name="c", subcore_axis_name="s")
    f = pl.kernel(
        _body,
        out_type=jax.ShapeDtypeStruct((_G, _SPAN, _CHUNK, _FDIM), jnp.float32),
        mesh=mesh,
        compiler_params=pltpu.CompilerParams(
            needs_layout_passes=False, use_tc_tiling_on_sc=False),
        scratch_types=[
            pltpu.VMEM((3, _SPAN, _CHUNK), jnp.int32),
            pltpu.VMEM((3, _SPAN, _CHUNK), jnp.int32),
            pltpu.VMEM((_SPAN, _CHUNK, _WDIM), jnp.float32),
            pltpu.VMEM((_SPAN, _CHUNK, _WDIM), jnp.float32),
            pltpu.VMEM((_SPAN, _CHUNK, 2 * _PDIM), jnp.float32),
            pltpu.VMEM((_SPAN, _CHUNK, 2 * _PDIM), jnp.float32),
            pltpu.VMEM((2 * _L * _PDIM,), jnp.float32),
            pltpu.VMEM((2 * _L * _PDIM,), jnp.float32),
            pltpu.SemaphoreType.DMA((2,)),
            pltpu.SemaphoreType.DMA((2,)),
        ],
    )
    out = f(idx, word_table,
            pos1_table.reshape(-1), pos2_table.reshape(-1))
    return out.reshape(_B, _L, _FDIM)
